# baseline (speedup 1.0000x reference)
import functools

import jax
import jax.numpy as jnp
from jax import lax
from jax.experimental import pallas as pl
from jax.experimental.pallas import tpu as pltpu

_EPS = 1e-3
_VMEM_LIMIT = 56 * 1024 * 1024


def _fold_bn(gamma, beta, mean, var):
    inv_std = 1.0 / jnp.sqrt(var.astype(jnp.float32) + _EPS)
    scale = gamma.astype(jnp.float32) * inv_std
    bias = beta.astype(jnp.float32) - mean.astype(jnp.float32) * scale
    return scale, bias


def _mat1x1(w, scale):
    return jnp.transpose(w[:, :, 0, 0]).astype(jnp.float32) * scale[None, :]


def _taps(w, scale, span):
    t = w[:, :, :, 0] if span == 'H' else w[:, :, 0, :]
    t = jnp.transpose(t, (2, 1, 0)).astype(jnp.float32)
    return t * scale[None, None, :]


def _inception_kernel(x_ref, wall_ref, b12_ref, b0_ref, b3_ref,
                      w11_ref, b11_ref, w21_ref, b21_ref,
                      w22_ref, b22_ref, w23_ref, b23_ref,
                      w1f_ref, b1f_ref, w2f_ref, b2f_ref, wm_ref, o_ref,
                      *, hh, ww, bn, c0, c1s, c2s, c1f, c2f, c3):
    hw = hh * ww
    rows = hw * bn
    f32 = jnp.float32
    bf16 = jnp.bfloat16

    xb = x_ref[...].reshape(rows, x_ref.shape[-1])

    acc = lax.dot_general(xb, wall_ref[...], (((1,), (0,)), ((), ())),
                          preferred_element_type=f32)
    stem12 = jnp.maximum(acc[:, :c1s + c2s] + b12_ref[...], 0.0).astype(bf16)
    x0 = jnp.maximum(acc[:, c1s + c2s:c1s + c2s + c0] + b0_ref[...], 0.0)
    x0 = x0.astype(bf16).astype(f32)
    o_ref[:, :, 0:c0] = x0.reshape(hw, bn, c0)
    y3 = acc[:, c1s + c2s + c0:]

    pi = lax.broadcasted_iota(jnp.int32, (rows, c3), 0) // bn
    wi = pi % ww
    hi = pi // ww
    zw = jnp.zeros((bn, c3), f32)
    s1 = jnp.concatenate([zw, y3, zw], axis=0)
    rowsum = (jnp.where(wi > 0, s1[0:rows, :], 0.0)
              + s1[bn:bn + rows, :]
              + jnp.where(wi < ww - 1, s1[2 * bn:2 * bn + rows, :], 0.0))
    zh = jnp.zeros((ww * bn, c3), f32)
    s2 = jnp.concatenate([zh, rowsum, zh], axis=0)
    colsum = (s2[0:rows, :] + s2[ww * bn:ww * bn + rows, :]
              + s2[2 * ww * bn:2 * ww * bn + rows, :])
    inv_w = jnp.where((wi == 0) | (wi == ww - 1), 0.5, 1.0 / 3.0)
    inv_h = jnp.where((hi == 0) | (hi == hh - 1), 0.5, 1.0 / 3.0)
    x3 = jnp.maximum(colsum * (inv_w * inv_h) + b3_ref[...], 0.0)
    o_ref[:, :, c0 + c1f + c2f:] = x3.reshape(hw, bn, c3)

    def tap(act, w_ref, wm_ref, b_ref, span, out_f32):
        cin = act.shape[1]
        stride = bn if span == 'W' else ww * bn
        z = jnp.zeros((3 * stride, cin), bf16)
        slab = jnp.concatenate([z, act, z], axis=0)
        acc_t = None
        for t in range(7):
            sl = slab[t * stride:t * stride + rows, :]
            if span == 'W' and t != 3:
                sl = sl * wm_ref[t][:, :cin]
            p = lax.dot_general(sl, w_ref[t], (((1,), (0,)), ((), ())),
                                preferred_element_type=f32)
            acc_t = p if acc_t is None else acc_t + p
        r = jnp.maximum(acc_t + b_ref[...], 0.0)
        return r if out_f32 else r.astype(bf16)

    a = tap(stem12[:, 0:c1s], w11_ref, wm_ref, b11_ref, 'W', False)
    x1 = tap(a, w1f_ref, None, b1f_ref, 'H', True)
    o_ref[:, :, c0:c0 + c1f] = x1.reshape(hw, bn, c1f)

    b = tap(stem12[:, c1s:], w21_ref, None, b21_ref, 'H', False)
    b = tap(b, w22_ref, wm_ref, b22_ref, 'W', False)
    b = tap(b, w23_ref, None, b23_ref, 'H', False)
    x2 = tap(b, w2f_ref, wm_ref, b2f_ref, 'W', True)
    o_ref[:, :, c0 + c1f:c0 + c1f + c2f] = x2.reshape(hw, bn, c2f)


def kernel(x,
           b0_0_w, b0_0_gamma, b0_0_beta, b0_0_mean, b0_0_var,
           b1_0_w, b1_0_gamma, b1_0_beta, b1_0_mean, b1_0_var,
           b1_1_w, b1_1_gamma, b1_1_beta, b1_1_mean, b1_1_var,
           b1_2_w, b1_2_gamma, b1_2_beta, b1_2_mean, b1_2_var,
           b2_0_w, b2_0_gamma, b2_0_beta, b2_0_mean, b2_0_var,
           b2_1_w, b2_1_gamma, b2_1_beta, b2_1_mean, b2_1_var,
           b2_2_w, b2_2_gamma, b2_2_beta, b2_2_mean, b2_2_var,
           b2_3_w, b2_3_gamma, b2_3_beta, b2_3_mean, b2_3_var,
           b2_4_w, b2_4_gamma, b2_4_beta, b2_4_mean, b2_4_var,
           b3_0_w, b3_0_gamma, b3_0_beta, b3_0_mean, b3_0_var):
    n, cin, hh, ww = x.shape
    hw = hh * ww
    bn = 8 if n % 8 == 0 else n
    bf16 = jnp.bfloat16
    f32 = jnp.float32

    s00, a00 = _fold_bn(b0_0_gamma, b0_0_beta, b0_0_mean, b0_0_var)
    s10, a10 = _fold_bn(b1_0_gamma, b1_0_beta, b1_0_mean, b1_0_var)
    s11, a11 = _fold_bn(b1_1_gamma, b1_1_beta, b1_1_mean, b1_1_var)
    s12, a12 = _fold_bn(b1_2_gamma, b1_2_beta, b1_2_mean, b1_2_var)
    s20, a20 = _fold_bn(b2_0_gamma, b2_0_beta, b2_0_mean, b2_0_var)
    s21, a21 = _fold_bn(b2_1_gamma, b2_1_beta, b2_1_mean, b2_1_var)
    s22, a22 = _fold_bn(b2_2_gamma, b2_2_beta, b2_2_mean, b2_2_var)
    s23, a23 = _fold_bn(b2_3_gamma, b2_3_beta, b2_3_mean, b2_3_var)
    s24, a24 = _fold_bn(b2_4_gamma, b2_4_beta, b2_4_mean, b2_4_var)
    s30, a30 = _fold_bn(b3_0_gamma, b3_0_beta, b3_0_mean, b3_0_var)

    c0 = b0_0_w.shape[0]
    c1s = b1_0_w.shape[0]
    c2s = b2_0_w.shape[0]
    c1f = b1_2_w.shape[0]
    c2f = b2_4_w.shape[0]
    c3 = b3_0_w.shape[0]
    ctot = c0 + c1f + c2f + c3

    wall = jnp.concatenate(
        [_mat1x1(b1_0_w, s10), _mat1x1(b2_0_w, s20),
         _mat1x1(b0_0_w, s00), _mat1x1(b3_0_w, s30)], axis=1).astype(bf16)
    b12 = jnp.concatenate([a10, a20]).reshape(1, c1s + c2s).astype(f32)
    b0b = a00.reshape(1, c0).astype(f32)
    b3b = a30.reshape(1, c3).astype(f32)

    w11 = _taps(b1_1_w, s11, 'W').astype(bf16)
    b11 = a11.reshape(1, -1).astype(f32)
    w21 = _taps(b2_1_w, s21, 'H').astype(bf16)
    b21 = a21.reshape(1, -1).astype(f32)
    w22 = _taps(b2_2_w, s22, 'W').astype(bf16)
    b22 = a22.reshape(1, -1).astype(f32)
    w23 = _taps(b2_3_w, s23, 'H').astype(bf16)
    b23 = a23.reshape(1, -1).astype(f32)
    w1f = _taps(b1_2_w, s12, 'H').astype(bf16)
    b1f = a12.reshape(1, -1).astype(f32)
    w2f = _taps(b2_4_w, s24, 'W').astype(bf16)
    b2f = a24.reshape(1, -1).astype(f32)

    x_p = jnp.transpose(x, (2, 3, 0, 1)).reshape(hw, n, cin).astype(bf16)

    rows = hw * bn
    cmax = max(b1_1_w.shape[1], b2_2_w.shape[1], b2_4_w.shape[1])
    wcol = (jnp.arange(rows, dtype=jnp.int32) // bn) % ww
    d = jnp.arange(7, dtype=jnp.int32) - 3
    valid = ((wcol[None, :] + d[:, None] >= 0)
             & (wcol[None, :] + d[:, None] < ww))
    wm = jnp.broadcast_to(
        jnp.where(valid, 1.0, 0.0).astype(bf16)[:, :, None], (7, rows, cmax))

    kfn = functools.partial(_inception_kernel, hh=hh, ww=ww, bn=bn, c0=c0,
                            c1s=c1s, c2s=c2s, c1f=c1f, c2f=c2f, c3=c3)
    const = lambda i: (0, 0)
    const3 = lambda i: (0, 0, 0)
    out = pl.pallas_call(
        kfn,
        out_shape=jax.ShapeDtypeStruct((hw, n, ctot), f32),
        grid_spec=pltpu.PrefetchScalarGridSpec(
            num_scalar_prefetch=0,
            grid=(n // bn,),
            in_specs=[
                pl.BlockSpec((hw, bn, cin), lambda i: (0, i, 0)),
                pl.BlockSpec(wall.shape, const),
                pl.BlockSpec(b12.shape, const),
                pl.BlockSpec(b0b.shape, const),
                pl.BlockSpec(b3b.shape, const),
                pl.BlockSpec(w11.shape, const3),
                pl.BlockSpec(b11.shape, const),
                pl.BlockSpec(w21.shape, const3),
                pl.BlockSpec(b21.shape, const),
                pl.BlockSpec(w22.shape, const3),
                pl.BlockSpec(b22.shape, const),
                pl.BlockSpec(w23.shape, const3),
                pl.BlockSpec(b23.shape, const),
                pl.BlockSpec(w1f.shape, const3),
                pl.BlockSpec(b1f.shape, const),
                pl.BlockSpec(w2f.shape, const3),
                pl.BlockSpec(b2f.shape, const),
                pl.BlockSpec(wm.shape, const3),
            ],
            out_specs=pl.BlockSpec((hw, bn, ctot), lambda i: (0, i, 0))),
        compiler_params=pltpu.CompilerParams(
            dimension_semantics=("parallel",),
            vmem_limit_bytes=_VMEM_LIMIT),
    )(x_p, wall, b12, b0b, b3b, w11, b11, w21, b21,
      w22, b22, w23, b23, w1f, b1f, w2f, b2f, wm)
    return jnp.transpose(out.reshape(hh, ww, n, ctot), (2, 3, 0, 1))


# speedup vs baseline: 1.2745x; 1.2745x over previous
import functools

import jax
import jax.numpy as jnp
from jax import lax
from jax.experimental import pallas as pl
from jax.experimental.pallas import tpu as pltpu

_EPS = 1e-3
_VMEM_LIMIT = 56 * 1024 * 1024


def _fold_bn(gamma, beta, mean, var):
    inv_std = 1.0 / jnp.sqrt(var.astype(jnp.float32) + _EPS)
    scale = gamma.astype(jnp.float32) * inv_std
    bias = beta.astype(jnp.float32) - mean.astype(jnp.float32) * scale
    return scale, bias


def _mat1x1(w, scale):
    return jnp.transpose(w[:, :, 0, 0]).astype(jnp.float32) * scale[None, :]


def _taps(w, scale, span):
    t = w[:, :, :, 0] if span == 'H' else w[:, :, 0, :]
    t = jnp.transpose(t, (2, 1, 0)).astype(jnp.float32)
    return t * scale[None, None, :]


def _inception_kernel(x_ref, wall_ref, b12_ref, b0_ref, b3_ref,
                      w11_ref, b11_ref, w21_ref, b21_ref,
                      w22_ref, b22_ref, w23_ref, b23_ref,
                      w1f_ref, b1f_ref, w2f_ref, b2f_ref, inv3_ref, o_ref,
                      *, hh, ww, bn, c0, c1s, c2s, c1f, c2f, c3):
    hw = hh * ww
    rows = hw * bn
    f32 = jnp.float32
    bf16 = jnp.bfloat16

    xb = x_ref[...].reshape(rows, x_ref.shape[-1])

    acc = lax.dot_general(xb, wall_ref[...], (((1,), (0,)), ((), ())),
                          preferred_element_type=f32)
    stem12 = jnp.maximum(acc[:, :c1s + c2s] + b12_ref[...], 0.0).astype(bf16)
    x0 = jnp.maximum(acc[:, c1s + c2s:c1s + c2s + c0] + b0_ref[...], 0.0)
    x0 = x0.astype(bf16).astype(f32)
    o_ref[:, :, 0:c0] = x0.reshape(hw, bn, c0)
    y3 = acc[:, c1s + c2s + c0:]

    y3r = y3.reshape(hh, ww * bn, c3)
    zw = jnp.zeros((hh, bn, c3), f32)
    s1 = jnp.concatenate([zw, y3r, zw], axis=1)
    rowsum = (s1[:, 0:ww * bn, :] + s1[:, bn:(ww + 1) * bn, :]
              + s1[:, 2 * bn:(ww + 2) * bn, :])
    zh = jnp.zeros((1, ww * bn, c3), f32)
    s2 = jnp.concatenate([zh, rowsum, zh], axis=0)
    colsum = (s2[0:hh] + s2[1:hh + 1] + s2[2:hh + 2]).reshape(rows, c3)
    x3 = jnp.maximum(colsum * inv3_ref[...] + b3_ref[...], 0.0)
    o_ref[:, :, c0 + c1f + c2f:] = x3.reshape(hw, bn, c3)

    def tap(act, w_ref, b_ref, span, out_f32):
        cin = act.shape[1]
        if span == 'W':
            a3 = act.reshape(hh, ww * bn, cin)
            zp = jnp.zeros((hh, 3 * bn, cin), bf16)
            slab = jnp.concatenate([zp, a3, zp], axis=1)
            slices = [
                slab[:, t * bn:(t + ww) * bn, :].reshape(rows, cin)
                for t in range(7)
            ]
        else:
            stride = ww * bn
            z = jnp.zeros((3 * stride, cin), bf16)
            slab = jnp.concatenate([z, act, z], axis=0)
            slices = [slab[t * stride:t * stride + rows, :] for t in range(7)]
        rchunk = 776
        outs = []
        for r0 in range(0, rows, rchunk):
            rc = min(rchunk, rows - r0)
            acc_t = None
            for t in range(7):
                p = lax.dot_general(slices[t][r0:r0 + rc, :], w_ref[t],
                                    (((1,), (0,)), ((), ())),
                                    preferred_element_type=f32)
                acc_t = p if acc_t is None else acc_t + p
            outs.append(jnp.maximum(acc_t + b_ref[...], 0.0))
        r = jnp.concatenate(outs, axis=0) if len(outs) > 1 else outs[0]
        return r if out_f32 else r.astype(bf16)

    a = tap(stem12[:, 0:c1s], w11_ref, b11_ref, 'W', False)
    x1 = tap(a, w1f_ref, b1f_ref, 'H', True)
    o_ref[:, :, c0:c0 + c1f] = x1.reshape(hw, bn, c1f)

    b = tap(stem12[:, c1s:], w21_ref, b21_ref, 'H', False)
    b = tap(b, w22_ref, b22_ref, 'W', False)
    b = tap(b, w23_ref, b23_ref, 'H', False)
    x2 = tap(b, w2f_ref, b2f_ref, 'W', True)
    o_ref[:, :, c0 + c1f:c0 + c1f + c2f] = x2.reshape(hw, bn, c2f)


def kernel(x,
           b0_0_w, b0_0_gamma, b0_0_beta, b0_0_mean, b0_0_var,
           b1_0_w, b1_0_gamma, b1_0_beta, b1_0_mean, b1_0_var,
           b1_1_w, b1_1_gamma, b1_1_beta, b1_1_mean, b1_1_var,
           b1_2_w, b1_2_gamma, b1_2_beta, b1_2_mean, b1_2_var,
           b2_0_w, b2_0_gamma, b2_0_beta, b2_0_mean, b2_0_var,
           b2_1_w, b2_1_gamma, b2_1_beta, b2_1_mean, b2_1_var,
           b2_2_w, b2_2_gamma, b2_2_beta, b2_2_mean, b2_2_var,
           b2_3_w, b2_3_gamma, b2_3_beta, b2_3_mean, b2_3_var,
           b2_4_w, b2_4_gamma, b2_4_beta, b2_4_mean, b2_4_var,
           b3_0_w, b3_0_gamma, b3_0_beta, b3_0_mean, b3_0_var):
    n, cin, hh, ww = x.shape
    hw = hh * ww
    bn = 8 if n % 8 == 0 else n
    bf16 = jnp.bfloat16
    f32 = jnp.float32

    s00, a00 = _fold_bn(b0_0_gamma, b0_0_beta, b0_0_mean, b0_0_var)
    s10, a10 = _fold_bn(b1_0_gamma, b1_0_beta, b1_0_mean, b1_0_var)
    s11, a11 = _fold_bn(b1_1_gamma, b1_1_beta, b1_1_mean, b1_1_var)
    s12, a12 = _fold_bn(b1_2_gamma, b1_2_beta, b1_2_mean, b1_2_var)
    s20, a20 = _fold_bn(b2_0_gamma, b2_0_beta, b2_0_mean, b2_0_var)
    s21, a21 = _fold_bn(b2_1_gamma, b2_1_beta, b2_1_mean, b2_1_var)
    s22, a22 = _fold_bn(b2_2_gamma, b2_2_beta, b2_2_mean, b2_2_var)
    s23, a23 = _fold_bn(b2_3_gamma, b2_3_beta, b2_3_mean, b2_3_var)
    s24, a24 = _fold_bn(b2_4_gamma, b2_4_beta, b2_4_mean, b2_4_var)
    s30, a30 = _fold_bn(b3_0_gamma, b3_0_beta, b3_0_mean, b3_0_var)

    c0 = b0_0_w.shape[0]
    c1s = b1_0_w.shape[0]
    c2s = b2_0_w.shape[0]
    c1f = b1_2_w.shape[0]
    c2f = b2_4_w.shape[0]
    c3 = b3_0_w.shape[0]
    ctot = c0 + c1f + c2f + c3

    wall = jnp.concatenate(
        [_mat1x1(b1_0_w, s10), _mat1x1(b2_0_w, s20),
         _mat1x1(b0_0_w, s00), _mat1x1(b3_0_w, s30)], axis=1).astype(bf16)
    b12 = jnp.concatenate([a10, a20]).reshape(1, c1s + c2s).astype(f32)
    b0b = a00.reshape(1, c0).astype(f32)
    b3b = a30.reshape(1, c3).astype(f32)

    w11 = _taps(b1_1_w, s11, 'W').astype(bf16)
    b11 = a11.reshape(1, -1).astype(f32)
    w21 = _taps(b2_1_w, s21, 'H').astype(bf16)
    b21 = a21.reshape(1, -1).astype(f32)
    w22 = _taps(b2_2_w, s22, 'W').astype(bf16)
    b22 = a22.reshape(1, -1).astype(f32)
    w23 = _taps(b2_3_w, s23, 'H').astype(bf16)
    b23 = a23.reshape(1, -1).astype(f32)
    w1f = _taps(b1_2_w, s12, 'H').astype(bf16)
    b1f = a12.reshape(1, -1).astype(f32)
    w2f = _taps(b2_4_w, s24, 'W').astype(bf16)
    b2f = a24.reshape(1, -1).astype(f32)

    x_p = jnp.transpose(x, (2, 3, 0, 1)).reshape(hw, n, cin).astype(bf16)

    ih = jnp.arange(hh, dtype=jnp.int32)
    iw = jnp.arange(ww, dtype=jnp.int32)
    cnt_h = jnp.minimum(ih + 1, hh - 1) - jnp.maximum(ih - 1, 0) + 1
    cnt_w = jnp.minimum(iw + 1, ww - 1) - jnp.maximum(iw - 1, 0) + 1
    inv = 1.0 / (cnt_h[:, None] * cnt_w[None, :]).astype(f32)
    inv3 = jnp.broadcast_to(
        inv.reshape(hw)[:, None, None], (hw, bn, c3)).reshape(hw * bn, c3)


    kfn = functools.partial(_inception_kernel, hh=hh, ww=ww, bn=bn, c0=c0,
                            c1s=c1s, c2s=c2s, c1f=c1f, c2f=c2f, c3=c3)
    const = lambda i: (0, 0)
    const3 = lambda i: (0, 0, 0)
    out = pl.pallas_call(
        kfn,
        out_shape=jax.ShapeDtypeStruct((hw, n, ctot), f32),
        grid_spec=pltpu.PrefetchScalarGridSpec(
            num_scalar_prefetch=0,
            grid=(n // bn,),
            in_specs=[
                pl.BlockSpec((hw, bn, cin), lambda i: (0, i, 0)),
                pl.BlockSpec(wall.shape, const),
                pl.BlockSpec(b12.shape, const),
                pl.BlockSpec(b0b.shape, const),
                pl.BlockSpec(b3b.shape, const),
                pl.BlockSpec(w11.shape, const3),
                pl.BlockSpec(b11.shape, const),
                pl.BlockSpec(w21.shape, const3),
                pl.BlockSpec(b21.shape, const),
                pl.BlockSpec(w22.shape, const3),
                pl.BlockSpec(b22.shape, const),
                pl.BlockSpec(w23.shape, const3),
                pl.BlockSpec(b23.shape, const),
                pl.BlockSpec(w1f.shape, const3),
                pl.BlockSpec(b1f.shape, const),
                pl.BlockSpec(w2f.shape, const3),
                pl.BlockSpec(b2f.shape, const),
                pl.BlockSpec(inv3.shape, const),
            ],
            out_specs=pl.BlockSpec((hw, bn, ctot), lambda i: (0, i, 0))),
        compiler_params=pltpu.CompilerParams(
            dimension_semantics=("parallel",),
            vmem_limit_bytes=_VMEM_LIMIT),
    )(x_p, wall, b12, b0b, b3b, w11, b11, w21, b21,
      w22, b22, w23, b23, w1f, b1f, w2f, b2f, inv3)
    return jnp.transpose(out.reshape(hh, ww, n, ctot), (2, 3, 0, 1))


# speedup vs baseline: 1.4628x; 1.1478x over previous
import functools

import jax
import jax.numpy as jnp
from jax import lax
from jax.experimental import pallas as pl
from jax.experimental.pallas import tpu as pltpu

_EPS = 1e-3
_VMEM_LIMIT = 58 * 1024 * 1024


def _fold_bn(gamma, beta, mean, var):
    inv_std = 1.0 / jnp.sqrt(var.astype(jnp.float32) + _EPS)
    scale = gamma.astype(jnp.float32) * inv_std
    bias = beta.astype(jnp.float32) - mean.astype(jnp.float32) * scale
    return scale, bias


def _mat1x1(w, scale):
    return jnp.transpose(w[:, :, 0, 0]).astype(jnp.float32) * scale[None, :]


def _taps(w, scale, span):
    t = w[:, :, :, 0] if span == 'H' else w[:, :, 0, :]
    t = jnp.transpose(t, (2, 1, 0)).astype(jnp.float32)
    return t * scale[None, None, :]


def _inception_kernel(x_ref, wall_ref, b12_ref, b0_ref, b3_ref,
                      w11_ref, b11_ref, w21_ref, b21_ref,
                      w22_ref, b22_ref, w23_ref, b23_ref,
                      w1f_ref, b1f_ref, w2f_ref, b2f_ref, inv3_ref, o_ref,
                      *, hh, ww, bn, c0, c1s, c2s, c1f, c2f, c3):
    hw = hh * ww
    rows = hw * bn
    f32 = jnp.float32
    bf16 = jnp.bfloat16

    xb = x_ref[...].astype(bf16).reshape(rows, x_ref.shape[-1])

    acc = lax.dot_general(xb, wall_ref[...], (((1,), (0,)), ((), ())),
                          preferred_element_type=f32)
    stem12 = jnp.maximum(acc[:, :c1s + c2s] + b12_ref[...], 0.0).astype(bf16)
    x0 = jnp.maximum(acc[:, c1s + c2s:c1s + c2s + c0] + b0_ref[...], 0.0)
    x0 = x0.astype(bf16).astype(f32)
    o_ref[:, :, 0:c0] = x0.reshape(hw, bn, c0)
    y3 = acc[:, c1s + c2s + c0:]

    y3r = y3.reshape(hh, ww * bn, c3)
    zw = jnp.zeros((hh, bn, c3), f32)
    s1 = jnp.concatenate([zw, y3r, zw], axis=1)
    rowsum = (s1[:, 0:ww * bn, :] + s1[:, bn:(ww + 1) * bn, :]
              + s1[:, 2 * bn:(ww + 2) * bn, :])
    zh = jnp.zeros((1, ww * bn, c3), f32)
    s2 = jnp.concatenate([zh, rowsum, zh], axis=0)
    colsum = (s2[0:hh] + s2[1:hh + 1] + s2[2:hh + 2]).reshape(rows, c3)
    x3 = jnp.maximum(colsum * inv3_ref[...] + b3_ref[...], 0.0)
    o_ref[:, :, c0 + c1f + c2f:] = x3.reshape(hw, bn, c3)

    def tap(act, w_ref, b_ref, span, out_f32):
        cin = act.shape[1]
        if span == 'W':
            a3 = act.reshape(hh, ww * bn, cin)
            zp = jnp.zeros((hh, 3 * bn, cin), bf16)
            slab = jnp.concatenate([zp, a3, zp], axis=1)
            slices = [
                slab[:, t * bn:(t + ww) * bn, :].reshape(rows, cin)
                for t in range(7)
            ]
        else:
            stride = ww * bn
            z = jnp.zeros((3 * stride, cin), bf16)
            slab = jnp.concatenate([z, act, z], axis=0)
            slices = [slab[t * stride:t * stride + rows, :] for t in range(7)]
        rchunk = 776
        outs = []
        for r0 in range(0, rows, rchunk):
            rc = min(rchunk, rows - r0)
            acc_t = None
            for t in range(7):
                p = lax.dot_general(slices[t][r0:r0 + rc, :], w_ref[t],
                                    (((1,), (0,)), ((), ())),
                                    preferred_element_type=f32)
                acc_t = p if acc_t is None else acc_t + p
            outs.append(jnp.maximum(acc_t + b_ref[...], 0.0))
        r = jnp.concatenate(outs, axis=0) if len(outs) > 1 else outs[0]
        return r if out_f32 else r.astype(bf16)

    a = tap(stem12[:, 0:c1s], w11_ref, b11_ref, 'W', False)
    x1 = tap(a, w1f_ref, b1f_ref, 'H', True)
    o_ref[:, :, c0:c0 + c1f] = x1.reshape(hw, bn, c1f)

    b = tap(stem12[:, c1s:], w21_ref, b21_ref, 'H', False)
    b = tap(b, w22_ref, b22_ref, 'W', False)
    b = tap(b, w23_ref, b23_ref, 'H', False)
    x2 = tap(b, w2f_ref, b2f_ref, 'W', True)
    o_ref[:, :, c0 + c1f:c0 + c1f + c2f] = x2.reshape(hw, bn, c2f)


def kernel(x,
           b0_0_w, b0_0_gamma, b0_0_beta, b0_0_mean, b0_0_var,
           b1_0_w, b1_0_gamma, b1_0_beta, b1_0_mean, b1_0_var,
           b1_1_w, b1_1_gamma, b1_1_beta, b1_1_mean, b1_1_var,
           b1_2_w, b1_2_gamma, b1_2_beta, b1_2_mean, b1_2_var,
           b2_0_w, b2_0_gamma, b2_0_beta, b2_0_mean, b2_0_var,
           b2_1_w, b2_1_gamma, b2_1_beta, b2_1_mean, b2_1_var,
           b2_2_w, b2_2_gamma, b2_2_beta, b2_2_mean, b2_2_var,
           b2_3_w, b2_3_gamma, b2_3_beta, b2_3_mean, b2_3_var,
           b2_4_w, b2_4_gamma, b2_4_beta, b2_4_mean, b2_4_var,
           b3_0_w, b3_0_gamma, b3_0_beta, b3_0_mean, b3_0_var):
    n, cin, hh, ww = x.shape
    hw = hh * ww
    bn = 8 if n % 8 == 0 else n
    bf16 = jnp.bfloat16
    f32 = jnp.float32

    s00, a00 = _fold_bn(b0_0_gamma, b0_0_beta, b0_0_mean, b0_0_var)
    s10, a10 = _fold_bn(b1_0_gamma, b1_0_beta, b1_0_mean, b1_0_var)
    s11, a11 = _fold_bn(b1_1_gamma, b1_1_beta, b1_1_mean, b1_1_var)
    s12, a12 = _fold_bn(b1_2_gamma, b1_2_beta, b1_2_mean, b1_2_var)
    s20, a20 = _fold_bn(b2_0_gamma, b2_0_beta, b2_0_mean, b2_0_var)
    s21, a21 = _fold_bn(b2_1_gamma, b2_1_beta, b2_1_mean, b2_1_var)
    s22, a22 = _fold_bn(b2_2_gamma, b2_2_beta, b2_2_mean, b2_2_var)
    s23, a23 = _fold_bn(b2_3_gamma, b2_3_beta, b2_3_mean, b2_3_var)
    s24, a24 = _fold_bn(b2_4_gamma, b2_4_beta, b2_4_mean, b2_4_var)
    s30, a30 = _fold_bn(b3_0_gamma, b3_0_beta, b3_0_mean, b3_0_var)

    c0 = b0_0_w.shape[0]
    c1s = b1_0_w.shape[0]
    c2s = b2_0_w.shape[0]
    c1f = b1_2_w.shape[0]
    c2f = b2_4_w.shape[0]
    c3 = b3_0_w.shape[0]
    ctot = c0 + c1f + c2f + c3

    wall = jnp.concatenate(
        [_mat1x1(b1_0_w, s10), _mat1x1(b2_0_w, s20),
         _mat1x1(b0_0_w, s00), _mat1x1(b3_0_w, s30)], axis=1).astype(bf16)
    b12 = jnp.concatenate([a10, a20]).reshape(1, c1s + c2s).astype(f32)
    b0b = a00.reshape(1, c0).astype(f32)
    b3b = a30.reshape(1, c3).astype(f32)

    w11 = _taps(b1_1_w, s11, 'W').astype(bf16)
    b11 = a11.reshape(1, -1).astype(f32)
    w21 = _taps(b2_1_w, s21, 'H').astype(bf16)
    b21 = a21.reshape(1, -1).astype(f32)
    w22 = _taps(b2_2_w, s22, 'W').astype(bf16)
    b22 = a22.reshape(1, -1).astype(f32)
    w23 = _taps(b2_3_w, s23, 'H').astype(bf16)
    b23 = a23.reshape(1, -1).astype(f32)
    w1f = _taps(b1_2_w, s12, 'H').astype(bf16)
    b1f = a12.reshape(1, -1).astype(f32)
    w2f = _taps(b2_4_w, s24, 'W').astype(bf16)
    b2f = a24.reshape(1, -1).astype(f32)

    x_p = jnp.transpose(x, (2, 3, 0, 1)).reshape(hw, n, cin)

    ih = jnp.arange(hh, dtype=jnp.int32)
    iw = jnp.arange(ww, dtype=jnp.int32)
    cnt_h = jnp.minimum(ih + 1, hh - 1) - jnp.maximum(ih - 1, 0) + 1
    cnt_w = jnp.minimum(iw + 1, ww - 1) - jnp.maximum(iw - 1, 0) + 1
    inv = 1.0 / (cnt_h[:, None] * cnt_w[None, :]).astype(f32)
    inv3 = jnp.broadcast_to(
        inv.reshape(hw)[:, None, None], (hw, bn, c3)).reshape(hw * bn, c3)


    kfn = functools.partial(_inception_kernel, hh=hh, ww=ww, bn=bn, c0=c0,
                            c1s=c1s, c2s=c2s, c1f=c1f, c2f=c2f, c3=c3)
    const = lambda i: (0, 0)
    const3 = lambda i: (0, 0, 0)
    out = pl.pallas_call(
        kfn,
        out_shape=jax.ShapeDtypeStruct((hw, n, ctot), f32),
        grid_spec=pltpu.PrefetchScalarGridSpec(
            num_scalar_prefetch=0,
            grid=(n // bn,),
            in_specs=[
                pl.BlockSpec((hw, bn, cin), lambda i: (0, i, 0)),
                pl.BlockSpec(wall.shape, const),
                pl.BlockSpec(b12.shape, const),
                pl.BlockSpec(b0b.shape, const),
                pl.BlockSpec(b3b.shape, const),
                pl.BlockSpec(w11.shape, const3),
                pl.BlockSpec(b11.shape, const),
                pl.BlockSpec(w21.shape, const3),
                pl.BlockSpec(b21.shape, const),
                pl.BlockSpec(w22.shape, const3),
                pl.BlockSpec(b22.shape, const),
                pl.BlockSpec(w23.shape, const3),
                pl.BlockSpec(b23.shape, const),
                pl.BlockSpec(w1f.shape, const3),
                pl.BlockSpec(b1f.shape, const),
                pl.BlockSpec(w2f.shape, const3),
                pl.BlockSpec(b2f.shape, const),
                pl.BlockSpec(inv3.shape, const),
            ],
            out_specs=pl.BlockSpec((hw, bn, ctot), lambda i: (0, i, 0))),
        compiler_params=pltpu.CompilerParams(
            dimension_semantics=("parallel",),
            vmem_limit_bytes=_VMEM_LIMIT),
    )(x_p, wall, b12, b0b, b3b, w11, b11, w21, b21,
      w22, b22, w23, b23, w1f, b1f, w2f, b2f, inv3)
    return jnp.transpose(out.reshape(hh, ww, n, ctot), (2, 3, 0, 1))


# speedup vs baseline: 1.5503x; 1.0598x over previous
import functools

import jax
import jax.numpy as jnp
from jax import lax
from jax.experimental import pallas as pl
from jax.experimental.pallas import tpu as pltpu

_EPS = 1e-3
_VMEM_LIMIT = 58 * 1024 * 1024


def _fold_bn(gamma, beta, mean, var):
    inv_std = 1.0 / jnp.sqrt(var.astype(jnp.float32) + _EPS)
    scale = gamma.astype(jnp.float32) * inv_std
    bias = beta.astype(jnp.float32) - mean.astype(jnp.float32) * scale
    return scale, bias


def _mat1x1(w, scale):
    return jnp.transpose(w[:, :, 0, 0]).astype(jnp.float32) * scale[None, :]


def _taps(w, scale, span):
    t = w[:, :, :, 0] if span == 'H' else w[:, :, 0, :]
    t = jnp.transpose(t, (2, 1, 0)).astype(jnp.float32)
    return t * scale[None, None, :]


def _inception_kernel(x_ref, wall_ref, ball_ref,
                      w11_ref, w21_ref, w22_ref, w23_ref,
                      w1f_ref, w2f_ref, inv3_ref, o_ref,
                      *, hh, ww, bn, c0, c1s, c2s, c1f, c2f, c3, offs):
    hw = hh * ww
    rows = hw * bn
    f32 = jnp.float32
    bf16 = jnp.bfloat16

    xb = x_ref[...].astype(bf16).reshape(rows, x_ref.shape[-1])

    def bias(i):
        return ball_ref[:, offs[i]:offs[i + 1]]

    acc = lax.dot_general(xb, wall_ref[...], (((1,), (1,)), ((), ())),
                          preferred_element_type=f32)
    c12 = c1s + c2s
    acc = acc + ball_ref[:, 0:c12 + c0 + c3]
    stem12 = jnp.maximum(acc[:, :c12], 0.0).astype(bf16)
    x0 = jnp.maximum(acc[:, c12:c12 + c0], 0.0)
    x0 = x0.astype(bf16).astype(f32)
    o_ref[:, :, 0:c0] = x0.reshape(hw, bn, c0)
    y3 = acc[:, c12 + c0:]

    y3r = y3.reshape(hh, ww * bn, c3)
    zw = jnp.zeros((hh, bn, c3), f32)
    s1 = jnp.concatenate([zw, y3r, zw], axis=1)
    rowsum = (s1[:, 0:ww * bn, :] + s1[:, bn:(ww + 1) * bn, :]
              + s1[:, 2 * bn:(ww + 2) * bn, :])
    zh = jnp.zeros((1, ww * bn, c3), f32)
    s2 = jnp.concatenate([zh, rowsum, zh], axis=0)
    colsum = (s2[0:hh] + s2[1:hh + 1] + s2[2:hh + 2]).reshape(rows, c3)
    x3 = jnp.maximum(colsum * inv3_ref[...], 0.0)
    o_ref[:, :, c0 + c1f + c2f:] = x3.reshape(hw, bn, c3)

    def tap(act, w_ref, b_row, span, out_f32):
        cin = act.shape[1]
        if span == 'W':
            a3 = act.reshape(hh, ww * bn, cin)
            zp = jnp.zeros((hh, 3 * bn, cin), bf16)
            slab = jnp.concatenate([zp, a3, zp], axis=1)
            slices = [
                slab[:, t * bn:(t + ww) * bn, :].reshape(rows, cin)
                for t in range(7)
            ]
        else:
            stride = ww * bn
            z = jnp.zeros((3 * stride, cin), bf16)
            slab = jnp.concatenate([z, act, z], axis=0)
            slices = [slab[t * stride:t * stride + rows, :] for t in range(7)]
        rchunk = 776
        outs = []
        for r0 in range(0, rows, rchunk):
            rc = min(rchunk, rows - r0)
            acc_t = None
            for t in range(7):
                p = lax.dot_general(slices[t][r0:r0 + rc, :], w_ref[t],
                                    (((1,), (0,)), ((), ())),
                                    preferred_element_type=f32)
                acc_t = p if acc_t is None else acc_t + p
            outs.append(jnp.maximum(acc_t + b_row, 0.0))
        r = jnp.concatenate(outs, axis=0) if len(outs) > 1 else outs[0]
        return r if out_f32 else r.astype(bf16)

    a = tap(stem12[:, 0:c1s], w11_ref, bias(4), 'W', False)
    x1 = tap(a, w1f_ref, bias(8), 'H', True)
    o_ref[:, :, c0:c0 + c1f] = x1.reshape(hw, bn, c1f)

    b = tap(stem12[:, c1s:], w21_ref, bias(5), 'H', False)
    b = tap(b, w22_ref, bias(6), 'W', False)
    b = tap(b, w23_ref, bias(7), 'H', False)
    x2 = tap(b, w2f_ref, bias(9), 'W', True)
    o_ref[:, :, c0 + c1f:c0 + c1f + c2f] = x2.reshape(hw, bn, c2f)


def kernel(x,
           b0_0_w, b0_0_gamma, b0_0_beta, b0_0_mean, b0_0_var,
           b1_0_w, b1_0_gamma, b1_0_beta, b1_0_mean, b1_0_var,
           b1_1_w, b1_1_gamma, b1_1_beta, b1_1_mean, b1_1_var,
           b1_2_w, b1_2_gamma, b1_2_beta, b1_2_mean, b1_2_var,
           b2_0_w, b2_0_gamma, b2_0_beta, b2_0_mean, b2_0_var,
           b2_1_w, b2_1_gamma, b2_1_beta, b2_1_mean, b2_1_var,
           b2_2_w, b2_2_gamma, b2_2_beta, b2_2_mean, b2_2_var,
           b2_3_w, b2_3_gamma, b2_3_beta, b2_3_mean, b2_3_var,
           b2_4_w, b2_4_gamma, b2_4_beta, b2_4_mean, b2_4_var,
           b3_0_w, b3_0_gamma, b3_0_beta, b3_0_mean, b3_0_var):
    n, cin, hh, ww = x.shape
    hw = hh * ww
    bn = 8 if n % 8 == 0 else n
    bf16 = jnp.bfloat16
    f32 = jnp.float32

    c0 = b0_0_w.shape[0]
    c1s = b1_0_w.shape[0]
    c2s = b2_0_w.shape[0]
    c1m = b1_1_w.shape[0]
    c2m1 = b2_1_w.shape[0]
    c2m2 = b2_2_w.shape[0]
    c2m3 = b2_3_w.shape[0]
    c1f = b1_2_w.shape[0]
    c2f = b2_4_w.shape[0]
    c3 = b3_0_w.shape[0]
    ctot = c0 + c1f + c2f + c3

    segs = [c1s, c2s, c0, c3, c1m, c2m1, c2m2, c2m3, c1f, c2f]
    parts = ['b1_0', 'b2_0', 'b0_0', 'b3_0',
             'b1_1', 'b2_1', 'b2_2', 'b2_3', 'b1_2', 'b2_4']
    prm = {
        'b0_0': (b0_0_gamma, b0_0_beta, b0_0_mean, b0_0_var),
        'b1_0': (b1_0_gamma, b1_0_beta, b1_0_mean, b1_0_var),
        'b1_1': (b1_1_gamma, b1_1_beta, b1_1_mean, b1_1_var),
        'b1_2': (b1_2_gamma, b1_2_beta, b1_2_mean, b1_2_var),
        'b2_0': (b2_0_gamma, b2_0_beta, b2_0_mean, b2_0_var),
        'b2_1': (b2_1_gamma, b2_1_beta, b2_1_mean, b2_1_var),
        'b2_2': (b2_2_gamma, b2_2_beta, b2_2_mean, b2_2_var),
        'b2_3': (b2_3_gamma, b2_3_beta, b2_3_mean, b2_3_var),
        'b2_4': (b2_4_gamma, b2_4_beta, b2_4_mean, b2_4_var),
        'b3_0': (b3_0_gamma, b3_0_beta, b3_0_mean, b3_0_var),
    }
    g_all = jnp.concatenate([prm[p][0] for p in parts])
    be_all = jnp.concatenate([prm[p][1] for p in parts])
    m_all = jnp.concatenate([prm[p][2] for p in parts])
    v_all = jnp.concatenate([prm[p][3] for p in parts])
    scale_all, bias_all = _fold_bn(g_all, be_all, m_all, v_all)
    offs = [0]
    for c in segs:
        offs.append(offs[-1] + c)

    def sseg(i):
        return scale_all[offs[i]:offs[i + 1]]

    wallT = jnp.concatenate(
        [b1_0_w[:, :, 0, 0] * sseg(0)[:, None],
         b2_0_w[:, :, 0, 0] * sseg(1)[:, None],
         b0_0_w[:, :, 0, 0] * sseg(2)[:, None],
         b3_0_w[:, :, 0, 0] * sseg(3)[:, None]], axis=0).astype(bf16)
    ball = bias_all.reshape(1, -1).astype(f32)

    w11 = _taps(b1_1_w, sseg(4), 'W').astype(bf16)
    w21 = _taps(b2_1_w, sseg(5), 'H').astype(bf16)
    w22 = _taps(b2_2_w, sseg(6), 'W').astype(bf16)
    w23 = _taps(b2_3_w, sseg(7), 'H').astype(bf16)
    w1f = _taps(b1_2_w, sseg(8), 'H').astype(bf16)
    w2f = _taps(b2_4_w, sseg(9), 'W').astype(bf16)

    x_p = jnp.transpose(x, (2, 3, 0, 1)).reshape(hw, n, cin)

    ih = jnp.arange(hh, dtype=jnp.int32)
    iw = jnp.arange(ww, dtype=jnp.int32)
    cnt_h = jnp.minimum(ih + 1, hh - 1) - jnp.maximum(ih - 1, 0) + 1
    cnt_w = jnp.minimum(iw + 1, ww - 1) - jnp.maximum(iw - 1, 0) + 1
    inv = 1.0 / (cnt_h[:, None] * cnt_w[None, :]).astype(f32)
    inv3 = jnp.broadcast_to(
        inv.reshape(hw)[:, None, None], (hw, bn, c3)).reshape(hw * bn, c3)


    kfn = functools.partial(_inception_kernel, hh=hh, ww=ww, bn=bn, c0=c0,
                            c1s=c1s, c2s=c2s, c1f=c1f, c2f=c2f, c3=c3,
                            offs=tuple(offs))
    const = lambda i: (0, 0)
    const3 = lambda i: (0, 0, 0)
    out = pl.pallas_call(
        kfn,
        out_shape=jax.ShapeDtypeStruct((hw, n, ctot), f32),
        grid_spec=pltpu.PrefetchScalarGridSpec(
            num_scalar_prefetch=0,
            grid=(n // bn,),
            in_specs=[
                pl.BlockSpec((hw, bn, cin), lambda i: (0, i, 0)),
                pl.BlockSpec(wallT.shape, const),
                pl.BlockSpec(ball.shape, const),
                pl.BlockSpec(w11.shape, const3),
                pl.BlockSpec(w21.shape, const3),
                pl.BlockSpec(w22.shape, const3),
                pl.BlockSpec(w23.shape, const3),
                pl.BlockSpec(w1f.shape, const3),
                pl.BlockSpec(w2f.shape, const3),
                pl.BlockSpec(inv3.shape, const),
            ],
            out_specs=pl.BlockSpec((hw, bn, ctot), lambda i: (0, i, 0))),
        compiler_params=pltpu.CompilerParams(
            dimension_semantics=("parallel",),
            vmem_limit_bytes=_VMEM_LIMIT),
    )(x_p, wallT, ball, w11, w21, w22, w23, w1f, w2f, inv3)
    return jnp.transpose(out.reshape(hh, ww, n, ctot), (2, 3, 0, 1))


# speedup vs baseline: 1.5515x; 1.0008x over previous
import functools

import jax
import jax.numpy as jnp
from jax import lax
from jax.experimental import pallas as pl
from jax.experimental.pallas import tpu as pltpu

_EPS = 1e-3
_VMEM_LIMIT = 58 * 1024 * 1024


def _fold_bn(gamma, beta, mean, var):
    inv_std = 1.0 / jnp.sqrt(var.astype(jnp.float32) + _EPS)
    scale = gamma.astype(jnp.float32) * inv_std
    bias = beta.astype(jnp.float32) - mean.astype(jnp.float32) * scale
    return scale, bias


def _mat1x1(w, scale):
    return jnp.transpose(w[:, :, 0, 0]).astype(jnp.float32) * scale[None, :]


def _taps(w, scale, span):
    t = w[:, :, :, 0] if span == 'H' else w[:, :, 0, :]
    t = jnp.transpose(t, (2, 1, 0)).astype(jnp.float32)
    return t * scale[None, None, :]


def _inception_kernel(x_ref, wall_ref, ball_ref,
                      w11_ref, w21_ref, w22_ref, w23_ref,
                      w1f_ref, w2f_ref, inv3_ref, o_ref,
                      *, hh, ww, bn, c0, c1s, c2s, c1f, c2f, c3, offs):
    hw = hh * ww
    rows = hw * bn
    f32 = jnp.float32
    bf16 = jnp.bfloat16

    xb = x_ref[...].astype(bf16).reshape(rows, x_ref.shape[-1])

    def bias(i):
        return ball_ref[:, offs[i]:offs[i + 1]]

    acc = lax.dot_general(xb, wall_ref[...], (((1,), (1,)), ((), ())),
                          preferred_element_type=f32)
    c12 = c1s + c2s
    acc = acc + ball_ref[:, 0:c12 + c0 + c3]
    stem12 = jnp.maximum(acc[:, :c12], 0.0).astype(bf16)
    x0 = jnp.maximum(acc[:, c12:c12 + c0], 0.0)
    x0 = x0.astype(bf16).astype(f32)
    o_ref[:, :, 0:c0] = x0.reshape(hw, bn, c0)
    y3 = acc[:, c12 + c0:]

    y3r = y3.reshape(hh, ww * bn, c3)
    zw = jnp.zeros((hh, bn, c3), f32)
    s1 = jnp.concatenate([zw, y3r, zw], axis=1)
    rowsum = (s1[:, 0:ww * bn, :] + s1[:, bn:(ww + 1) * bn, :]
              + s1[:, 2 * bn:(ww + 2) * bn, :])
    zh = jnp.zeros((1, ww * bn, c3), f32)
    s2 = jnp.concatenate([zh, rowsum, zh], axis=0)
    colsum = (s2[0:hh] + s2[1:hh + 1] + s2[2:hh + 2]).reshape(rows, c3)
    x3 = jnp.maximum(colsum * inv3_ref[...], 0.0)
    o_ref[:, :, c0 + c1f + c2f:] = x3.reshape(hw, bn, c3)

    def tap(act, w_ref, b_row, span, out_f32):
        cin = act.shape[1]
        if span == 'W':
            a3 = act.reshape(hh, ww * bn, cin)
            zp = jnp.zeros((hh, 3 * bn, cin), bf16)
            slab = jnp.concatenate([zp, a3, zp], axis=1)
            slices = [
                slab[:, t * bn:(t + ww) * bn, :].reshape(rows, cin)
                for t in range(7)
            ]
        else:
            stride = ww * bn
            z = jnp.zeros((3 * stride, cin), bf16)
            slab = jnp.concatenate([z, act, z], axis=0)
            slices = [slab[t * stride:t * stride + rows, :] for t in range(7)]
        rchunk = 776
        outs = []
        for r0 in range(0, rows, rchunk):
            rc = min(rchunk, rows - r0)
            acc_t = None
            for t in range(7):
                p = lax.dot_general(slices[t][r0:r0 + rc, :], w_ref[t],
                                    (((1,), (0,)), ((), ())),
                                    preferred_element_type=f32)
                acc_t = p if acc_t is None else acc_t + p
            outs.append(jnp.maximum(acc_t + b_row, 0.0))
        r = jnp.concatenate(outs, axis=0) if len(outs) > 1 else outs[0]
        return r if out_f32 else r.astype(bf16)

    a = tap(stem12[:, 0:c1s], w11_ref, bias(4), 'W', False)
    x1 = tap(a, w1f_ref, bias(8), 'H', True)
    o_ref[:, :, c0:c0 + c1f] = x1.reshape(hw, bn, c1f)

    b = tap(stem12[:, c1s:], w21_ref, bias(5), 'H', False)
    b = tap(b, w22_ref, bias(6), 'W', False)
    b = tap(b, w23_ref, bias(7), 'H', False)
    x2 = tap(b, w2f_ref, bias(9), 'W', True)
    o_ref[:, :, c0 + c1f:c0 + c1f + c2f] = x2.reshape(hw, bn, c2f)


def kernel(x,
           b0_0_w, b0_0_gamma, b0_0_beta, b0_0_mean, b0_0_var,
           b1_0_w, b1_0_gamma, b1_0_beta, b1_0_mean, b1_0_var,
           b1_1_w, b1_1_gamma, b1_1_beta, b1_1_mean, b1_1_var,
           b1_2_w, b1_2_gamma, b1_2_beta, b1_2_mean, b1_2_var,
           b2_0_w, b2_0_gamma, b2_0_beta, b2_0_mean, b2_0_var,
           b2_1_w, b2_1_gamma, b2_1_beta, b2_1_mean, b2_1_var,
           b2_2_w, b2_2_gamma, b2_2_beta, b2_2_mean, b2_2_var,
           b2_3_w, b2_3_gamma, b2_3_beta, b2_3_mean, b2_3_var,
           b2_4_w, b2_4_gamma, b2_4_beta, b2_4_mean, b2_4_var,
           b3_0_w, b3_0_gamma, b3_0_beta, b3_0_mean, b3_0_var):
    n, cin, hh, ww = x.shape
    hw = hh * ww
    bn = 8 if n % 8 == 0 else n
    bf16 = jnp.bfloat16
    f32 = jnp.float32

    c0 = b0_0_w.shape[0]
    c1s = b1_0_w.shape[0]
    c2s = b2_0_w.shape[0]
    c1m = b1_1_w.shape[0]
    c2m1 = b2_1_w.shape[0]
    c2m2 = b2_2_w.shape[0]
    c2m3 = b2_3_w.shape[0]
    c1f = b1_2_w.shape[0]
    c2f = b2_4_w.shape[0]
    c3 = b3_0_w.shape[0]
    ctot = c0 + c1f + c2f + c3

    segs = [c1s, c2s, c0, c3, c1m, c2m1, c2m2, c2m3, c1f, c2f]
    parts = ['b1_0', 'b2_0', 'b0_0', 'b3_0',
             'b1_1', 'b2_1', 'b2_2', 'b2_3', 'b1_2', 'b2_4']
    prm = {
        'b0_0': (b0_0_gamma, b0_0_beta, b0_0_mean, b0_0_var),
        'b1_0': (b1_0_gamma, b1_0_beta, b1_0_mean, b1_0_var),
        'b1_1': (b1_1_gamma, b1_1_beta, b1_1_mean, b1_1_var),
        'b1_2': (b1_2_gamma, b1_2_beta, b1_2_mean, b1_2_var),
        'b2_0': (b2_0_gamma, b2_0_beta, b2_0_mean, b2_0_var),
        'b2_1': (b2_1_gamma, b2_1_beta, b2_1_mean, b2_1_var),
        'b2_2': (b2_2_gamma, b2_2_beta, b2_2_mean, b2_2_var),
        'b2_3': (b2_3_gamma, b2_3_beta, b2_3_mean, b2_3_var),
        'b2_4': (b2_4_gamma, b2_4_beta, b2_4_mean, b2_4_var),
        'b3_0': (b3_0_gamma, b3_0_beta, b3_0_mean, b3_0_var),
    }
    g_all = jnp.concatenate([prm[p][0] for p in parts])
    be_all = jnp.concatenate([prm[p][1] for p in parts])
    m_all = jnp.concatenate([prm[p][2] for p in parts])
    v_all = jnp.concatenate([prm[p][3] for p in parts])
    scale_all, bias_all = _fold_bn(g_all, be_all, m_all, v_all)
    offs = [0]
    for c in segs:
        offs.append(offs[-1] + c)

    def sseg(i):
        return scale_all[offs[i]:offs[i + 1]]

    wallT = jnp.concatenate(
        [b1_0_w[:, :, 0, 0] * sseg(0)[:, None],
         b2_0_w[:, :, 0, 0] * sseg(1)[:, None],
         b0_0_w[:, :, 0, 0] * sseg(2)[:, None],
         b3_0_w[:, :, 0, 0] * sseg(3)[:, None]], axis=0).astype(bf16)
    ball = bias_all.reshape(1, -1).astype(f32)

    w11 = _taps(b1_1_w, sseg(4), 'W').astype(bf16)
    w21 = _taps(b2_1_w, sseg(5), 'H').astype(bf16)
    w22 = _taps(b2_2_w, sseg(6), 'W').astype(bf16)
    w23 = _taps(b2_3_w, sseg(7), 'H').astype(bf16)
    w1f = _taps(b1_2_w, sseg(8), 'H').astype(bf16)
    w2f = _taps(b2_4_w, sseg(9), 'W').astype(bf16)

    x_p = jnp.transpose(x, (2, 3, 0, 1)).reshape(hw, n, cin)

    ih = jnp.arange(hh, dtype=jnp.int32)
    iw = jnp.arange(ww, dtype=jnp.int32)
    cnt_h = jnp.minimum(ih + 1, hh - 1) - jnp.maximum(ih - 1, 0) + 1
    cnt_w = jnp.minimum(iw + 1, ww - 1) - jnp.maximum(iw - 1, 0) + 1
    inv = 1.0 / (cnt_h[:, None] * cnt_w[None, :]).astype(f32)
    inv3 = jnp.broadcast_to(
        inv.reshape(hw)[:, None, None], (hw, bn, c3)).reshape(hw * bn, c3)


    kfn = functools.partial(_inception_kernel, hh=hh, ww=ww, bn=bn, c0=c0,
                            c1s=c1s, c2s=c2s, c1f=c1f, c2f=c2f, c3=c3,
                            offs=tuple(offs))
    const = lambda i: (0, 0)
    const3 = lambda i: (0, 0, 0)
    out = pl.pallas_call(
        kfn,
        out_shape=jax.ShapeDtypeStruct((hw, n, ctot), f32),
        grid_spec=pltpu.PrefetchScalarGridSpec(
            num_scalar_prefetch=0,
            grid=(n // bn,),
            in_specs=[
                pl.BlockSpec((hw, bn, cin), lambda i: (0, i, 0)),
                pl.BlockSpec(wallT.shape, const),
                pl.BlockSpec(ball.shape, const),
                pl.BlockSpec(w11.shape, const3),
                pl.BlockSpec(w21.shape, const3),
                pl.BlockSpec(w22.shape, const3),
                pl.BlockSpec(w23.shape, const3),
                pl.BlockSpec(w1f.shape, const3),
                pl.BlockSpec(w2f.shape, const3),
                pl.BlockSpec(inv3.shape, const),
            ],
            out_specs=pl.BlockSpec((hw, bn, ctot), lambda i: (0, i, 0))),
        compiler_params=pltpu.CompilerParams(
            dimension_semantics=("parallel",),
            vmem_limit_bytes=_VMEM_LIMIT,
            flags={"XLA_TPU_STORE_TO_LOAD_FORWARDING_WINDOW": 12288}),
    )(x_p, wallT, ball, w11, w21, w22, w23, w1f, w2f, inv3)
    return jnp.transpose(out.reshape(hh, ww, n, ctot), (2, 3, 0, 1))


# speedup vs baseline: 1.5556x; 1.0026x over previous
import functools

import jax
import jax.numpy as jnp
from jax import lax
from jax.experimental import pallas as pl
from jax.experimental.pallas import tpu as pltpu

_EPS = 1e-3
_VMEM_LIMIT = 58 * 1024 * 1024


def _fold_bn(gamma, beta, mean, var):
    inv_std = 1.0 / jnp.sqrt(var.astype(jnp.float32) + _EPS)
    scale = gamma.astype(jnp.float32) * inv_std
    bias = beta.astype(jnp.float32) - mean.astype(jnp.float32) * scale
    return scale, bias


def _taps(w, scale, span):
    t = w[:, :, :, 0] if span == 'H' else w[:, :, 0, :]
    t = jnp.transpose(t, (2, 1, 0)).astype(jnp.float32)
    return t * scale[None, None, :]


def _inception_kernel(x_ref, wall_ref, ball_ref,
                      w11_ref, w21_ref, w22_ref, w23_ref,
                      w1f_ref, w2f_ref, inv3_ref, o_ref,
                      *, hh, ww, bn, c0, c1s, c2s, c1f, c2f, c3, offs):
    hw = hh * ww
    rows = hw * bn
    f32 = jnp.float32
    bf16 = jnp.bfloat16

    xb = x_ref[...].astype(bf16).reshape(rows, x_ref.shape[-1])

    def bias(i):
        return ball_ref[:, offs[i]:offs[i + 1]]

    acc = lax.dot_general(xb, wall_ref[...], (((1,), (1,)), ((), ())),
                          preferred_element_type=f32)
    c12 = c1s + c2s
    acc = acc + ball_ref[:, 0:c12 + c0 + c3]
    stem12 = jnp.maximum(acc[:, :c12], 0.0).astype(bf16)
    x0 = jnp.maximum(acc[:, c12:c12 + c0], 0.0)
    x0 = x0.astype(bf16).astype(f32)
    o_ref[:, :, 0:c0] = x0.reshape(hw, bn, c0)
    y3 = acc[:, c12 + c0:]

    y3r = y3.reshape(hh, ww * bn, c3)
    zw = jnp.zeros((hh, bn, c3), f32)
    s1 = jnp.concatenate([zw, y3r, zw], axis=1)
    rowsum = (s1[:, 0:ww * bn, :] + s1[:, bn:(ww + 1) * bn, :]
              + s1[:, 2 * bn:(ww + 2) * bn, :])
    zh = jnp.zeros((1, ww * bn, c3), f32)
    s2 = jnp.concatenate([zh, rowsum, zh], axis=0)
    colsum = (s2[0:hh] + s2[1:hh + 1] + s2[2:hh + 2]).reshape(rows, c3)
    x3 = jnp.maximum(colsum * inv3_ref[...], 0.0)
    o_ref[:, :, c0 + c1f + c2f:] = x3.reshape(hw, bn, c3)

    def tap(act, w_ref, b_row, span, out_f32):
        cin = act.shape[1]
        if span == 'W':
            a3 = act.reshape(hh, ww * bn, cin)
            zp = jnp.zeros((hh, 3 * bn, cin), bf16)
            slab = jnp.concatenate([zp, a3, zp], axis=1)
            slices = [
                slab[:, t * bn:(t + ww) * bn, :].reshape(rows, cin)
                for t in range(7)
            ]
        else:
            stride = ww * bn
            z = jnp.zeros((3 * stride, cin), bf16)
            slab = jnp.concatenate([z, act, z], axis=0)
            slices = [slab[t * stride:t * stride + rows, :] for t in range(7)]
        rchunk = 776
        outs = []
        for r0 in range(0, rows, rchunk):
            rc = min(rchunk, rows - r0)
            acc_t = None
            for t in range(7):
                p = lax.dot_general(slices[t][r0:r0 + rc, :], w_ref[t],
                                    (((1,), (0,)), ((), ())),
                                    preferred_element_type=f32)
                acc_t = p if acc_t is None else acc_t + p
            outs.append(jnp.maximum(acc_t + b_row, 0.0))
        r = jnp.concatenate(outs, axis=0) if len(outs) > 1 else outs[0]
        return r if out_f32 else r.astype(bf16)

    a = tap(stem12[:, 0:c1s], w11_ref, bias(4), 'W', False)
    x1 = tap(a, w1f_ref, bias(8), 'H', True)
    o_ref[:, :, c0:c0 + c1f] = x1.reshape(hw, bn, c1f)

    b = tap(stem12[:, c1s:], w21_ref, bias(5), 'H', False)
    b = tap(b, w22_ref, bias(6), 'W', False)
    b = tap(b, w23_ref, bias(7), 'H', False)
    x2 = tap(b, w2f_ref, bias(9), 'W', True)
    o_ref[:, :, c0 + c1f:c0 + c1f + c2f] = x2.reshape(hw, bn, c2f)


def kernel(x,
           b0_0_w, b0_0_gamma, b0_0_beta, b0_0_mean, b0_0_var,
           b1_0_w, b1_0_gamma, b1_0_beta, b1_0_mean, b1_0_var,
           b1_1_w, b1_1_gamma, b1_1_beta, b1_1_mean, b1_1_var,
           b1_2_w, b1_2_gamma, b1_2_beta, b1_2_mean, b1_2_var,
           b2_0_w, b2_0_gamma, b2_0_beta, b2_0_mean, b2_0_var,
           b2_1_w, b2_1_gamma, b2_1_beta, b2_1_mean, b2_1_var,
           b2_2_w, b2_2_gamma, b2_2_beta, b2_2_mean, b2_2_var,
           b2_3_w, b2_3_gamma, b2_3_beta, b2_3_mean, b2_3_var,
           b2_4_w, b2_4_gamma, b2_4_beta, b2_4_mean, b2_4_var,
           b3_0_w, b3_0_gamma, b3_0_beta, b3_0_mean, b3_0_var):
    n, cin, hh, ww = x.shape
    hw = hh * ww
    bn = 8 if n % 8 == 0 else n
    bf16 = jnp.bfloat16
    f32 = jnp.float32

    c0 = b0_0_w.shape[0]
    c1s = b1_0_w.shape[0]
    c2s = b2_0_w.shape[0]
    c1m = b1_1_w.shape[0]
    c2m1 = b2_1_w.shape[0]
    c2m2 = b2_2_w.shape[0]
    c2m3 = b2_3_w.shape[0]
    c1f = b1_2_w.shape[0]
    c2f = b2_4_w.shape[0]
    c3 = b3_0_w.shape[0]
    ctot = c0 + c1f + c2f + c3

    segs = [c1s, c2s, c0, c3, c1m, c2m1, c2m2, c2m3, c1f, c2f]
    parts = ['b1_0', 'b2_0', 'b0_0', 'b3_0',
             'b1_1', 'b2_1', 'b2_2', 'b2_3', 'b1_2', 'b2_4']
    prm = {
        'b0_0': (b0_0_gamma, b0_0_beta, b0_0_mean, b0_0_var),
        'b1_0': (b1_0_gamma, b1_0_beta, b1_0_mean, b1_0_var),
        'b1_1': (b1_1_gamma, b1_1_beta, b1_1_mean, b1_1_var),
        'b1_2': (b1_2_gamma, b1_2_beta, b1_2_mean, b1_2_var),
        'b2_0': (b2_0_gamma, b2_0_beta, b2_0_mean, b2_0_var),
        'b2_1': (b2_1_gamma, b2_1_beta, b2_1_mean, b2_1_var),
        'b2_2': (b2_2_gamma, b2_2_beta, b2_2_mean, b2_2_var),
        'b2_3': (b2_3_gamma, b2_3_beta, b2_3_mean, b2_3_var),
        'b2_4': (b2_4_gamma, b2_4_beta, b2_4_mean, b2_4_var),
        'b3_0': (b3_0_gamma, b3_0_beta, b3_0_mean, b3_0_var),
    }
    g_all = jnp.concatenate([prm[p][0] for p in parts])
    be_all = jnp.concatenate([prm[p][1] for p in parts])
    m_all = jnp.concatenate([prm[p][2] for p in parts])
    v_all = jnp.concatenate([prm[p][3] for p in parts])
    scale_all, bias_all = _fold_bn(g_all, be_all, m_all, v_all)
    offs = [0]
    for c in segs:
        offs.append(offs[-1] + c)

    def sseg(i):
        return scale_all[offs[i]:offs[i + 1]]

    wallT = jnp.concatenate(
        [b1_0_w[:, :, 0, 0] * sseg(0)[:, None],
         b2_0_w[:, :, 0, 0] * sseg(1)[:, None],
         b0_0_w[:, :, 0, 0] * sseg(2)[:, None],
         b3_0_w[:, :, 0, 0] * sseg(3)[:, None]], axis=0).astype(bf16)
    ball = bias_all.reshape(1, -1).astype(f32)

    w11 = _taps(b1_1_w, sseg(4), 'W').astype(bf16)
    w21 = _taps(b2_1_w, sseg(5), 'H').astype(bf16)
    w22 = _taps(b2_2_w, sseg(6), 'W').astype(bf16)
    w23 = _taps(b2_3_w, sseg(7), 'H').astype(bf16)
    w1f = _taps(b1_2_w, sseg(8), 'H').astype(bf16)
    w2f = _taps(b2_4_w, sseg(9), 'W').astype(bf16)

    x_p = jnp.transpose(x, (2, 3, 0, 1)).reshape(hw, n, cin)

    ih = jnp.arange(hh, dtype=jnp.int32)
    iw = jnp.arange(ww, dtype=jnp.int32)
    cnt_h = jnp.minimum(ih + 1, hh - 1) - jnp.maximum(ih - 1, 0) + 1
    cnt_w = jnp.minimum(iw + 1, ww - 1) - jnp.maximum(iw - 1, 0) + 1
    inv = 1.0 / (cnt_h[:, None] * cnt_w[None, :]).astype(f32)
    inv3 = jnp.broadcast_to(
        inv.reshape(hw)[:, None, None], (hw, bn, c3)).reshape(hw * bn, c3)


    kfn = functools.partial(_inception_kernel, hh=hh, ww=ww, bn=bn, c0=c0,
                            c1s=c1s, c2s=c2s, c1f=c1f, c2f=c2f, c3=c3,
                            offs=tuple(offs))
    const = lambda i: (0, 0)
    const3 = lambda i: (0, 0, 0)
    out = pl.pallas_call(
        kfn,
        out_shape=jax.ShapeDtypeStruct((hw, n, ctot), f32),
        grid_spec=pltpu.PrefetchScalarGridSpec(
            num_scalar_prefetch=0,
            grid=(n // bn,),
            in_specs=[
                pl.BlockSpec((hw, bn, cin), lambda i: (0, i, 0)),
                pl.BlockSpec(wallT.shape, const),
                pl.BlockSpec(ball.shape, const),
                pl.BlockSpec(w11.shape, const3),
                pl.BlockSpec(w21.shape, const3),
                pl.BlockSpec(w22.shape, const3),
                pl.BlockSpec(w23.shape, const3),
                pl.BlockSpec(w1f.shape, const3),
                pl.BlockSpec(w2f.shape, const3),
                pl.BlockSpec(inv3.shape, const),
            ],
            out_specs=pl.BlockSpec((hw, bn, ctot), lambda i: (0, i, 0))),
        compiler_params=pltpu.CompilerParams(
            dimension_semantics=("parallel",),
            vmem_limit_bytes=_VMEM_LIMIT),
    )(x_p, wallT, ball, w11, w21, w22, w23, w1f, w2f, inv3)
    return jnp.transpose(out.reshape(hh, ww, n, ctot), (2, 3, 0, 1))
